# baseline (speedup 1.0000x reference)
import functools

import jax
import jax.numpy as jnp
from jax import lax
from jax.experimental import pallas as pl
from jax.experimental.pallas import tpu as pltpu
from jax.experimental.pallas import tpu_sc as plsc

N_NODES = 10000
N_EDGES = 320000
D = 128
H = 8
HD = 16

NC = 2
NS = 16
NW = NC * NS
E_PAD = 327680
EPW = E_PAD // NW
CH = 1024
CH_ROWS = CH // 128
SUB = 128
NSUB = CH // SUB
NCHUNK = EPW // CH
K0 = 6
K1 = 20 - K0
OFF1 = NS * K0 * CH
NR = 624


EB = E_PAD // 40


def _prep_body(x_ref, wn_ref, bn_ref, ea_ref, we_ref, be_ref, ei_ref,
               xp_ref, gates_ref, src_ref, tgt_ref):
    i = pl.program_id(0)

    @pl.when(i == 0)
    def _proj():
        xp_ref[pl.ds(0, N_NODES), :] = (
            jnp.dot(x_ref[...], wn_ref[...],
                    preferred_element_type=jnp.float32) + bn_ref[...])
        xp_ref[pl.ds(N_NODES, 8), :] = jnp.zeros((8, D), jnp.float32)

    z = jnp.dot(ea_ref[...], we_ref[...], preferred_element_type=jnp.float32)
    g = jax.nn.sigmoid(z + be_ref[...])
    gates_ref[...] = g

    col = i * EB + jax.lax.broadcasted_iota(jnp.int32, (2, EB), 1)
    ei = ei_ref[...]
    src_ref[...] = jnp.where(col[0] < N_EDGES, ei[0], N_NODES).reshape(
        EB // 128, 128)
    tgt_ref[...] = jnp.where(col[1] < N_EDGES, ei[1], 0).reshape(
        EB // 128, 128)


def _sum_body(a_ref, b_ref, out_ref):
    out_ref[...] = a_ref[...] + b_ref[...]


_GDN = lax.GatherDimensionNumbers(
    offset_dims=(), collapsed_slice_dims=(0,), start_index_map=(0,))


def _splat(gv, zero16, h):
    return lax.gather(gv, (zero16 + h).reshape(16, 1), _GDN, (1,),
                      mode=lax.GatherScatterMode.PROMISE_IN_BOUNDS)


def _sc_body(xproj_hbm, src_hbm, tgt_hbm, gates_hbm, out_hbm,
             acc, src_v, tgt_v, gates_v, rows_a, rows_b,
             gsem0, gsem1, ssem0, ssem1, zsem):
    c = lax.axis_index("c")
    s = lax.axis_index("s")
    rows = (rows_a, rows_b)
    zero16 = lax.iota(jnp.int32, 16) * 0
    gsem = (gsem0, gsem1)
    ssem = (ssem0, ssem1)

    zf = jnp.zeros((16,), jnp.float32)

    def zero_rows(i, carry):
        for j in range(H):
            rows_a[i, pl.ds(j * HD, HD)] = zf
        return carry

    lax.fori_loop(0, SUB, zero_rows, 0)
    r0 = pl.multiple_of(s * NR, 8)
    descs = []
    for i in range(4):
        descs.append(pltpu.async_copy(
            rows_a, acc.at[pl.ds(r0 + i * SUB, SUB)], zsem))
    descs.append(pltpu.async_copy(
        rows_a.at[pl.ds(0, NR - 4 * SUB)],
        acc.at[pl.ds(r0 + 4 * SUB, NR - 4 * SUB)], zsem))
    for d in descs:
        d.wait()

    @pl.when(s == NS - 1)
    def _zero_tail():
        pltpu.async_copy(rows_a.at[pl.ds(0, 16)],
                         acc.at[pl.ds(NS * NR, 16)], zsem).wait()

    plsc.subcore_barrier()

    base0 = jnp.where(c == 0, s * (K0 * CH), OFF1 + s * (K1 * CH))
    nchunk = jnp.where(c == 0, K0, K1)

    def chunk_body(k, carry):
        base = pl.multiple_of(base0 + k * CH, CH)
        pltpu.sync_copy(
            src_hbm.at[pl.ds(pl.multiple_of(base // 128, CH_ROWS), CH_ROWS)],
            src_v)
        pltpu.sync_copy(
            tgt_hbm.at[pl.ds(pl.multiple_of(base // 128, CH_ROWS), CH_ROWS)],
            tgt_v)
        pltpu.sync_copy(
            gates_hbm.at[pl.ds(pl.multiple_of(base * H, CH * H), CH * H)],
            gates_v)

        gd = [None, None]
        sd = [None, None]
        gd[0] = pltpu.async_copy(
            xproj_hbm.at[src_v.at[0]], rows[0], gsem[0])
        for g in range(NSUB):
            b = g % 2
            nb = 1 - b
            if g < NSUB - 1:
                if sd[nb] is not None:
                    sd[nb].wait()
                gd[nb] = pltpu.async_copy(
                    xproj_hbm.at[src_v.at[g + 1]], rows[nb], gsem[nb])
            gd[b].wait()
            goff = g * SUB * H

            def pair_body(p, carry2, _b=b, _goff=goff):
                gv = gates_v[pl.ds(_goff + p * 16, 16)]
                e0 = 2 * p
                for h in range(H):
                    g0 = _splat(gv, zero16, h)
                    g1 = _splat(gv, zero16, h + H)
                    rows[_b][e0, pl.ds(h * HD, HD)] = (
                        rows[_b][e0, pl.ds(h * HD, HD)] * g0)
                    rows[_b][e0 + 1, pl.ds(h * HD, HD)] = (
                        rows[_b][e0 + 1, pl.ds(h * HD, HD)] * g1)
                return carry2

            lax.fori_loop(0, SUB // 2, pair_body, 0, unroll=2)
            sd[b] = pltpu.async_copy(rows[b], acc.at[tgt_v.at[g]],
                                     ssem[b], add=True)
        sd[0].wait()
        sd[1].wait()
        return carry

    lax.fori_loop(0, nchunk, chunk_body, 0)
    plsc.subcore_barrier()
    pltpu.sync_copy(acc.at[pl.ds(r0, NR)], out_hbm.at[c, pl.ds(r0, NR)])

    @pl.when(s == NS - 1)
    def _write_tail():
        pltpu.sync_copy(acc.at[pl.ds(NS * NR, 16)],
                        out_hbm.at[c, pl.ds(NS * NR, 16)])


def _make_sc_call():
    return functools.partial(
        pl.kernel,
        out_type=jax.ShapeDtypeStruct((NC, N_NODES, D), jnp.float32),
        mesh=plsc.VectorSubcoreMesh(core_axis_name="c", subcore_axis_name="s",
                                num_cores=NC, num_subcores=NS),
        scratch_types=[
        pltpu.VMEM_SHARED((N_NODES, D), jnp.float32),
        pltpu.VMEM((CH_ROWS, 128), jnp.int32),
        pltpu.VMEM((CH_ROWS, 128), jnp.int32),
        pltpu.VMEM((CH * H,), jnp.float32),
        pltpu.VMEM((SUB, D), jnp.float32),
        pltpu.VMEM((SUB, D), jnp.float32),
        pltpu.SemaphoreType.DMA,
        pltpu.SemaphoreType.DMA,
        pltpu.SemaphoreType.DMA,
        pltpu.SemaphoreType.DMA,
        pltpu.SemaphoreType.DMA,
        ],
    )(_sc_body)


_SC_CALL_CACHE = []


def _sc_call(*args):
    if not _SC_CALL_CACHE:
        _SC_CALL_CACHE.append(_make_sc_call())
    return _SC_CALL_CACHE[0](*args)


def kernel(x, edge_index, edge_attr, Wn, bn, We, be):
    x_proj, gates_p, src2, tgt2 = pl.pallas_call(
        _prep_body,
        grid=(40,),
        in_specs=[
            pl.BlockSpec((N_NODES, D), lambda i: (0, 0)),
            pl.BlockSpec((D, D), lambda i: (0, 0)),
            pl.BlockSpec((1, D), lambda i: (0, 0)),
            pl.BlockSpec((EB, 16), lambda i: (i, 0)),
            pl.BlockSpec((16, H), lambda i: (0, 0)),
            pl.BlockSpec((1, H), lambda i: (0, 0)),
            pl.BlockSpec((2, EB), lambda i: (0, i)),
        ],
        out_specs=[
            pl.BlockSpec((N_NODES + 8, D), lambda i: (0, 0)),
            pl.BlockSpec((EB, H), lambda i: (i, 0)),
            pl.BlockSpec((EB // 128, 128), lambda i: (i, 0)),
            pl.BlockSpec((EB // 128, 128), lambda i: (i, 0)),
        ],
        out_shape=[
            jax.ShapeDtypeStruct((N_NODES + 8, D), jnp.float32),
            jax.ShapeDtypeStruct((E_PAD, H), jnp.float32),
            jax.ShapeDtypeStruct((E_PAD // 128, 128), jnp.int32),
            jax.ShapeDtypeStruct((E_PAD // 128, 128), jnp.int32),
        ],
    )(x, Wn, bn.reshape(1, D), edge_attr, We, be.reshape(1, H),
      edge_index.astype(jnp.int32))

    parts = _sc_call(x_proj, src2, tgt2, gates_p.reshape(E_PAD * H))

    out = pl.pallas_call(
        _sum_body,
        out_shape=jax.ShapeDtypeStruct((N_NODES, D), jnp.float32),
    )(parts[0], parts[1])
    return out


# speedup vs baseline: 1.1524x; 1.1524x over previous
import functools

import jax
import jax.numpy as jnp
from jax import lax
from jax.experimental import pallas as pl
from jax.experimental.pallas import tpu as pltpu
from jax.experimental.pallas import tpu_sc as plsc

N_NODES = 10000
N_EDGES = 320000
D = 128
H = 8
HD = 16

NC = 2
NS = 16
NW = NC * NS
E_PAD = 327680
EPW = E_PAD // NW
CH = 1024
CH_ROWS = CH // 128
SUB = 128
NSUB = CH // SUB
NCHUNK = EPW // CH
K0 = 14
K1 = 20 - K0
OFF1 = NS * K0 * CH
NR = 624


EB = E_PAD // 40


def _prep_body(x_ref, wn_ref, bn_ref, ea_ref, we_ref, be_ref, ei_ref,
               xp_ref, gates_ref, src_ref, tgt_ref):
    i = pl.program_id(0)

    @pl.when(i == 0)
    def _proj():
        xp_ref[pl.ds(0, N_NODES), :] = (
            jnp.dot(x_ref[...], wn_ref[...],
                    preferred_element_type=jnp.float32) + bn_ref[...])
        xp_ref[pl.ds(N_NODES, 8), :] = jnp.zeros((8, D), jnp.float32)

    z = jnp.dot(ea_ref[...], we_ref[...], preferred_element_type=jnp.float32)
    g = jax.nn.sigmoid(z + be_ref[...])
    gates_ref[...] = g

    col = i * EB + jax.lax.broadcasted_iota(jnp.int32, (2, EB), 1)
    ei = ei_ref[...]
    src_ref[...] = jnp.where(col[0] < N_EDGES, ei[0], N_NODES).reshape(
        EB // 128, 128)
    tgt_ref[...] = jnp.where(col[1] < N_EDGES, ei[1], 0).reshape(
        EB // 128, 128)


def _sum_body(a_ref, b_ref, out_ref):
    out_ref[...] = a_ref[...] + b_ref[...]


_GDN = lax.GatherDimensionNumbers(
    offset_dims=(), collapsed_slice_dims=(0,), start_index_map=(0,))


def _splat(gv, zero16, h):
    return lax.gather(gv, (zero16 + h).reshape(16, 1), _GDN, (1,),
                      mode=lax.GatherScatterMode.PROMISE_IN_BOUNDS)


def _sc_body(xproj_hbm, src_hbm, tgt_hbm, gates_hbm, out_hbm,
             acc, src_v, tgt_v, gates_v, rows_a, rows_b,
             gsem0, gsem1, ssem0, ssem1, zsem):
    c = lax.axis_index("c")
    s = lax.axis_index("s")
    rows = (rows_a, rows_b)
    zero16 = lax.iota(jnp.int32, 16) * 0
    gsem = (gsem0, gsem1)
    ssem = (ssem0, ssem1)

    zf = jnp.zeros((16,), jnp.float32)

    def zero_rows(i, carry):
        for j in range(H):
            rows_a[i, pl.ds(j * HD, HD)] = zf
        return carry

    lax.fori_loop(0, SUB, zero_rows, 0)
    r0 = pl.multiple_of(s * NR, 8)
    descs = []
    for i in range(4):
        descs.append(pltpu.async_copy(
            rows_a, acc.at[pl.ds(r0 + i * SUB, SUB)], zsem))
    descs.append(pltpu.async_copy(
        rows_a.at[pl.ds(0, NR - 4 * SUB)],
        acc.at[pl.ds(r0 + 4 * SUB, NR - 4 * SUB)], zsem))
    for d in descs:
        d.wait()

    @pl.when(s == NS - 1)
    def _zero_tail():
        pltpu.async_copy(rows_a.at[pl.ds(0, 16)],
                         acc.at[pl.ds(NS * NR, 16)], zsem).wait()

    plsc.subcore_barrier()

    base0 = jnp.where(c == 0, s * (K0 * CH), OFF1 + s * (K1 * CH))
    nchunk = jnp.where(c == 0, K0, K1)

    def chunk_body(k, carry):
        base = pl.multiple_of(base0 + k * CH, CH)
        pltpu.sync_copy(
            src_hbm.at[pl.ds(pl.multiple_of(base // 128, CH_ROWS), CH_ROWS)],
            src_v)
        pltpu.sync_copy(
            tgt_hbm.at[pl.ds(pl.multiple_of(base // 128, CH_ROWS), CH_ROWS)],
            tgt_v)
        pltpu.sync_copy(
            gates_hbm.at[pl.ds(pl.multiple_of(base * H, CH * H), CH * H)],
            gates_v)

        gd = [None, None]
        sd = [None, None]
        gd[0] = pltpu.async_copy(
            xproj_hbm.at[src_v.at[0]], rows[0], gsem[0])
        for g in range(NSUB):
            b = g % 2
            nb = 1 - b
            if g < NSUB - 1:
                if sd[nb] is not None:
                    sd[nb].wait()
                gd[nb] = pltpu.async_copy(
                    xproj_hbm.at[src_v.at[g + 1]], rows[nb], gsem[nb])
            gd[b].wait()
            goff = g * SUB * H

            def pair_body(p, carry2, _b=b, _goff=goff):
                gv = gates_v[pl.ds(_goff + p * 16, 16)]
                e0 = 2 * p
                for h in range(H):
                    g0 = _splat(gv, zero16, h)
                    g1 = _splat(gv, zero16, h + H)
                    rows[_b][e0, pl.ds(h * HD, HD)] = (
                        rows[_b][e0, pl.ds(h * HD, HD)] * g0)
                    rows[_b][e0 + 1, pl.ds(h * HD, HD)] = (
                        rows[_b][e0 + 1, pl.ds(h * HD, HD)] * g1)
                return carry2

            lax.fori_loop(0, SUB // 2, pair_body, 0, unroll=2)
            sd[b] = pltpu.async_copy(rows[b], acc.at[tgt_v.at[g]],
                                     ssem[b], add=True)
        sd[0].wait()
        sd[1].wait()
        return carry

    lax.fori_loop(0, nchunk, chunk_body, 0)
    plsc.subcore_barrier()
    pltpu.sync_copy(acc.at[pl.ds(r0, NR)], out_hbm.at[c, pl.ds(r0, NR)])

    @pl.when(s == NS - 1)
    def _write_tail():
        pltpu.sync_copy(acc.at[pl.ds(NS * NR, 16)],
                        out_hbm.at[c, pl.ds(NS * NR, 16)])


def _make_sc_call():
    return functools.partial(
        pl.kernel,
        out_type=jax.ShapeDtypeStruct((NC, N_NODES, D), jnp.float32),
        mesh=plsc.VectorSubcoreMesh(core_axis_name="c", subcore_axis_name="s",
                                num_cores=NC, num_subcores=NS),
        scratch_types=[
        pltpu.VMEM_SHARED((N_NODES, D), jnp.float32),
        pltpu.VMEM((CH_ROWS, 128), jnp.int32),
        pltpu.VMEM((CH_ROWS, 128), jnp.int32),
        pltpu.VMEM((CH * H,), jnp.float32),
        pltpu.VMEM((SUB, D), jnp.float32),
        pltpu.VMEM((SUB, D), jnp.float32),
        pltpu.SemaphoreType.DMA,
        pltpu.SemaphoreType.DMA,
        pltpu.SemaphoreType.DMA,
        pltpu.SemaphoreType.DMA,
        pltpu.SemaphoreType.DMA,
        ],
    )(_sc_body)


_SC_CALL_CACHE = []


def _sc_call(*args):
    if not _SC_CALL_CACHE:
        _SC_CALL_CACHE.append(_make_sc_call())
    return _SC_CALL_CACHE[0](*args)


def kernel(x, edge_index, edge_attr, Wn, bn, We, be):
    x_proj, gates_p, src2, tgt2 = pl.pallas_call(
        _prep_body,
        grid=(40,),
        in_specs=[
            pl.BlockSpec((N_NODES, D), lambda i: (0, 0)),
            pl.BlockSpec((D, D), lambda i: (0, 0)),
            pl.BlockSpec((1, D), lambda i: (0, 0)),
            pl.BlockSpec((EB, 16), lambda i: (i, 0)),
            pl.BlockSpec((16, H), lambda i: (0, 0)),
            pl.BlockSpec((1, H), lambda i: (0, 0)),
            pl.BlockSpec((2, EB), lambda i: (0, i)),
        ],
        out_specs=[
            pl.BlockSpec((N_NODES + 8, D), lambda i: (0, 0)),
            pl.BlockSpec((EB, H), lambda i: (i, 0)),
            pl.BlockSpec((EB // 128, 128), lambda i: (i, 0)),
            pl.BlockSpec((EB // 128, 128), lambda i: (i, 0)),
        ],
        out_shape=[
            jax.ShapeDtypeStruct((N_NODES + 8, D), jnp.float32),
            jax.ShapeDtypeStruct((E_PAD, H), jnp.float32),
            jax.ShapeDtypeStruct((E_PAD // 128, 128), jnp.int32),
            jax.ShapeDtypeStruct((E_PAD // 128, 128), jnp.int32),
        ],
    )(x, Wn, bn.reshape(1, D), edge_attr, We, be.reshape(1, H),
      edge_index.astype(jnp.int32))

    parts = _sc_call(x_proj, src2, tgt2, gates_p.reshape(E_PAD * H))

    out = pl.pallas_call(
        _sum_body,
        out_shape=jax.ShapeDtypeStruct((N_NODES, D), jnp.float32),
    )(parts[0], parts[1])
    return out


# speedup vs baseline: 1.1630x; 1.0091x over previous
import functools

import jax
import jax.numpy as jnp
from jax import lax
from jax.experimental import pallas as pl
from jax.experimental.pallas import tpu as pltpu
from jax.experimental.pallas import tpu_sc as plsc

N_NODES = 10000
N_EDGES = 320000
D = 128
H = 8
HD = 16

NC = 2
NS = 16
NW = NC * NS
E_PAD = 327680
EPW = E_PAD // NW
CH = 1024
CH_ROWS = CH // 128
SUB = 128
NSUB = CH // SUB
NCHUNK = EPW // CH
K0 = 15
K1 = 20 - K0
OFF1 = NS * K0 * CH
NR = 624


EB = E_PAD // 40


def _prep_body(x_ref, wn_ref, bn_ref, ea_ref, we_ref, be_ref, ei_ref,
               xp_ref, gates_ref, src_ref, tgt_ref):
    i = pl.program_id(0)

    @pl.when(i == 0)
    def _proj():
        xp_ref[pl.ds(0, N_NODES), :] = (
            jnp.dot(x_ref[...], wn_ref[...],
                    preferred_element_type=jnp.float32) + bn_ref[...])
        xp_ref[pl.ds(N_NODES, 8), :] = jnp.zeros((8, D), jnp.float32)

    z = jnp.dot(ea_ref[...], we_ref[...], preferred_element_type=jnp.float32)
    g = jax.nn.sigmoid(z + be_ref[...])
    gates_ref[...] = g

    col = i * EB + jax.lax.broadcasted_iota(jnp.int32, (2, EB), 1)
    ei = ei_ref[...]
    src_ref[...] = jnp.where(col[0] < N_EDGES, ei[0], N_NODES).reshape(
        EB // 128, 128)
    tgt_ref[...] = jnp.where(col[1] < N_EDGES, ei[1], 0).reshape(
        EB // 128, 128)


def _sum_body(a_ref, b_ref, out_ref):
    out_ref[...] = a_ref[...] + b_ref[...]


_GDN = lax.GatherDimensionNumbers(
    offset_dims=(), collapsed_slice_dims=(0,), start_index_map=(0,))


def _splat(gv, zero16, h):
    return lax.gather(gv, (zero16 + h).reshape(16, 1), _GDN, (1,),
                      mode=lax.GatherScatterMode.PROMISE_IN_BOUNDS)


def _sc_body(xproj_hbm, src_hbm, tgt_hbm, gates_hbm, out_hbm,
             acc, src_v, tgt_v, gates_v, rows_a, rows_b,
             gsem0, gsem1, ssem0, ssem1, zsem):
    c = lax.axis_index("c")
    s = lax.axis_index("s")
    rows = (rows_a, rows_b)
    zero16 = lax.iota(jnp.int32, 16) * 0
    gsem = (gsem0, gsem1)
    ssem = (ssem0, ssem1)

    zf = jnp.zeros((16,), jnp.float32)

    def zero_rows(i, carry):
        for j in range(H):
            rows_a[i, pl.ds(j * HD, HD)] = zf
        return carry

    lax.fori_loop(0, SUB, zero_rows, 0)
    r0 = pl.multiple_of(s * NR, 8)
    descs = []
    for i in range(4):
        descs.append(pltpu.async_copy(
            rows_a, acc.at[pl.ds(r0 + i * SUB, SUB)], zsem))
    descs.append(pltpu.async_copy(
        rows_a.at[pl.ds(0, NR - 4 * SUB)],
        acc.at[pl.ds(r0 + 4 * SUB, NR - 4 * SUB)], zsem))
    for d in descs:
        d.wait()

    @pl.when(s == NS - 1)
    def _zero_tail():
        pltpu.async_copy(rows_a.at[pl.ds(0, 16)],
                         acc.at[pl.ds(NS * NR, 16)], zsem).wait()

    plsc.subcore_barrier()

    base0 = jnp.where(c == 0, s * (K0 * CH), OFF1 + s * (K1 * CH))
    nchunk = jnp.where(c == 0, K0, K1)

    def chunk_body(k, carry):
        base = pl.multiple_of(base0 + k * CH, CH)
        pltpu.sync_copy(
            src_hbm.at[pl.ds(pl.multiple_of(base // 128, CH_ROWS), CH_ROWS)],
            src_v)
        pltpu.sync_copy(
            tgt_hbm.at[pl.ds(pl.multiple_of(base // 128, CH_ROWS), CH_ROWS)],
            tgt_v)
        pltpu.sync_copy(
            gates_hbm.at[pl.ds(pl.multiple_of(base * H, CH * H), CH * H)],
            gates_v)

        gd = [None, None]
        sd = [None, None]
        gd[0] = pltpu.async_copy(
            xproj_hbm.at[src_v.at[0]], rows[0], gsem[0])
        for g in range(NSUB):
            b = g % 2
            nb = 1 - b
            if g < NSUB - 1:
                if sd[nb] is not None:
                    sd[nb].wait()
                gd[nb] = pltpu.async_copy(
                    xproj_hbm.at[src_v.at[g + 1]], rows[nb], gsem[nb])
            gd[b].wait()
            goff = g * SUB * H

            def pair_body(p, carry2, _b=b, _goff=goff):
                gv = gates_v[pl.ds(_goff + p * 16, 16)]
                e0 = 2 * p
                for h in range(H):
                    g0 = _splat(gv, zero16, h)
                    g1 = _splat(gv, zero16, h + H)
                    rows[_b][e0, pl.ds(h * HD, HD)] = (
                        rows[_b][e0, pl.ds(h * HD, HD)] * g0)
                    rows[_b][e0 + 1, pl.ds(h * HD, HD)] = (
                        rows[_b][e0 + 1, pl.ds(h * HD, HD)] * g1)
                return carry2

            lax.fori_loop(0, SUB // 2, pair_body, 0, unroll=2)
            sd[b] = pltpu.async_copy(rows[b], acc.at[tgt_v.at[g]],
                                     ssem[b], add=True)
        sd[0].wait()
        sd[1].wait()
        return carry

    lax.fori_loop(0, nchunk, chunk_body, 0)
    plsc.subcore_barrier()
    pltpu.sync_copy(acc.at[pl.ds(r0, NR)], out_hbm.at[c, pl.ds(r0, NR)])

    @pl.when(s == NS - 1)
    def _write_tail():
        pltpu.sync_copy(acc.at[pl.ds(NS * NR, 16)],
                        out_hbm.at[c, pl.ds(NS * NR, 16)])


def _make_sc_call():
    return functools.partial(
        pl.kernel,
        out_type=jax.ShapeDtypeStruct((NC, N_NODES, D), jnp.float32),
        mesh=plsc.VectorSubcoreMesh(core_axis_name="c", subcore_axis_name="s",
                                num_cores=NC, num_subcores=NS),
        scratch_types=[
        pltpu.VMEM_SHARED((N_NODES, D), jnp.float32),
        pltpu.VMEM((CH_ROWS, 128), jnp.int32),
        pltpu.VMEM((CH_ROWS, 128), jnp.int32),
        pltpu.VMEM((CH * H,), jnp.float32),
        pltpu.VMEM((SUB, D), jnp.float32),
        pltpu.VMEM((SUB, D), jnp.float32),
        pltpu.SemaphoreType.DMA,
        pltpu.SemaphoreType.DMA,
        pltpu.SemaphoreType.DMA,
        pltpu.SemaphoreType.DMA,
        pltpu.SemaphoreType.DMA,
        ],
    )(_sc_body)


_SC_CALL_CACHE = []


def _sc_call(*args):
    if not _SC_CALL_CACHE:
        _SC_CALL_CACHE.append(_make_sc_call())
    return _SC_CALL_CACHE[0](*args)


def kernel(x, edge_index, edge_attr, Wn, bn, We, be):
    x_proj, gates_p, src2, tgt2 = pl.pallas_call(
        _prep_body,
        grid=(40,),
        in_specs=[
            pl.BlockSpec((N_NODES, D), lambda i: (0, 0)),
            pl.BlockSpec((D, D), lambda i: (0, 0)),
            pl.BlockSpec((1, D), lambda i: (0, 0)),
            pl.BlockSpec((EB, 16), lambda i: (i, 0)),
            pl.BlockSpec((16, H), lambda i: (0, 0)),
            pl.BlockSpec((1, H), lambda i: (0, 0)),
            pl.BlockSpec((2, EB), lambda i: (0, i)),
        ],
        out_specs=[
            pl.BlockSpec((N_NODES + 8, D), lambda i: (0, 0)),
            pl.BlockSpec((EB, H), lambda i: (i, 0)),
            pl.BlockSpec((EB // 128, 128), lambda i: (i, 0)),
            pl.BlockSpec((EB // 128, 128), lambda i: (i, 0)),
        ],
        out_shape=[
            jax.ShapeDtypeStruct((N_NODES + 8, D), jnp.float32),
            jax.ShapeDtypeStruct((E_PAD, H), jnp.float32),
            jax.ShapeDtypeStruct((E_PAD // 128, 128), jnp.int32),
            jax.ShapeDtypeStruct((E_PAD // 128, 128), jnp.int32),
        ],
    )(x, Wn, bn.reshape(1, D), edge_attr, We, be.reshape(1, H),
      edge_index.astype(jnp.int32))

    parts = _sc_call(x_proj, src2, tgt2, gates_p.reshape(E_PAD * H))

    out = pl.pallas_call(
        _sum_body,
        out_shape=jax.ShapeDtypeStruct((N_NODES, D), jnp.float32),
    )(parts[0], parts[1])
    return out


# speedup vs baseline: 1.1872x; 1.0209x over previous
import functools

import jax
import jax.numpy as jnp
from jax import lax
from jax.experimental import pallas as pl
from jax.experimental.pallas import tpu as pltpu
from jax.experimental.pallas import tpu_sc as plsc

N_NODES = 10000
N_EDGES = 320000
D = 128
H = 8
HD = 16

NC = 2
NS = 16
NW = NC * NS
E_PAD = 327680
EPW = E_PAD // NW
CH = 1024
CH_ROWS = CH // 128
SUB = 128
NSUB = CH // SUB
NCHUNK = EPW // CH
K0 = 16
K1 = 20 - K0
OFF1 = NS * K0 * CH
NR = 624


EB = E_PAD // 40


def _prep_body(x_ref, wn_ref, bn_ref, ea_ref, we_ref, be_ref, ei_ref,
               xp_ref, gates_ref, src_ref, tgt_ref):
    i = pl.program_id(0)

    @pl.when(i == 0)
    def _proj():
        xp_ref[pl.ds(0, N_NODES), :] = (
            jnp.dot(x_ref[...], wn_ref[...],
                    preferred_element_type=jnp.float32) + bn_ref[...])
        xp_ref[pl.ds(N_NODES, 8), :] = jnp.zeros((8, D), jnp.float32)

    z = jnp.dot(ea_ref[...], we_ref[...], preferred_element_type=jnp.float32)
    g = jax.nn.sigmoid(z + be_ref[...])
    gates_ref[...] = g

    col = i * EB + jax.lax.broadcasted_iota(jnp.int32, (2, EB), 1)
    ei = ei_ref[...]
    src_ref[...] = jnp.where(col[0] < N_EDGES, ei[0], N_NODES).reshape(
        EB // 128, 128)
    tgt_ref[...] = jnp.where(col[1] < N_EDGES, ei[1], 0).reshape(
        EB // 128, 128)


def _sum_body(a_ref, b_ref, out_ref):
    out_ref[...] = a_ref[...] + b_ref[...]


_GDN = lax.GatherDimensionNumbers(
    offset_dims=(), collapsed_slice_dims=(0,), start_index_map=(0,))


def _splat(gv, zero16, h):
    return lax.gather(gv, (zero16 + h).reshape(16, 1), _GDN, (1,),
                      mode=lax.GatherScatterMode.PROMISE_IN_BOUNDS)


def _sc_body(xproj_hbm, src_hbm, tgt_hbm, gates_hbm, out_hbm,
             acc, src_v, tgt_v, gates_v, rows_a, rows_b,
             gsem0, gsem1, ssem0, ssem1, zsem):
    c = lax.axis_index("c")
    s = lax.axis_index("s")
    rows = (rows_a, rows_b)
    zero16 = lax.iota(jnp.int32, 16) * 0
    gsem = (gsem0, gsem1)
    ssem = (ssem0, ssem1)

    zf = jnp.zeros((16,), jnp.float32)

    def zero_rows(i, carry):
        for j in range(H):
            rows_a[i, pl.ds(j * HD, HD)] = zf
        return carry

    lax.fori_loop(0, SUB, zero_rows, 0)
    r0 = pl.multiple_of(s * NR, 8)
    descs = []
    for i in range(4):
        descs.append(pltpu.async_copy(
            rows_a, acc.at[pl.ds(r0 + i * SUB, SUB)], zsem))
    descs.append(pltpu.async_copy(
        rows_a.at[pl.ds(0, NR - 4 * SUB)],
        acc.at[pl.ds(r0 + 4 * SUB, NR - 4 * SUB)], zsem))
    for d in descs:
        d.wait()

    @pl.when(s == NS - 1)
    def _zero_tail():
        pltpu.async_copy(rows_a.at[pl.ds(0, 16)],
                         acc.at[pl.ds(NS * NR, 16)], zsem).wait()

    plsc.subcore_barrier()

    base0 = jnp.where(c == 0, s * (K0 * CH), OFF1 + s * (K1 * CH))
    nchunk = jnp.where(c == 0, K0, K1)

    def chunk_body(k, carry):
        base = pl.multiple_of(base0 + k * CH, CH)
        pltpu.sync_copy(
            src_hbm.at[pl.ds(pl.multiple_of(base // 128, CH_ROWS), CH_ROWS)],
            src_v)
        pltpu.sync_copy(
            tgt_hbm.at[pl.ds(pl.multiple_of(base // 128, CH_ROWS), CH_ROWS)],
            tgt_v)
        pltpu.sync_copy(
            gates_hbm.at[pl.ds(pl.multiple_of(base * H, CH * H), CH * H)],
            gates_v)

        gd = [None, None]
        sd = [None, None]
        gd[0] = pltpu.async_copy(
            xproj_hbm.at[src_v.at[0]], rows[0], gsem[0])
        for g in range(NSUB):
            b = g % 2
            nb = 1 - b
            if g < NSUB - 1:
                if sd[nb] is not None:
                    sd[nb].wait()
                gd[nb] = pltpu.async_copy(
                    xproj_hbm.at[src_v.at[g + 1]], rows[nb], gsem[nb])
            gd[b].wait()
            goff = g * SUB * H

            def pair_body(p, carry2, _b=b, _goff=goff):
                gv = gates_v[pl.ds(_goff + p * 16, 16)]
                e0 = 2 * p
                for h in range(H):
                    g0 = _splat(gv, zero16, h)
                    g1 = _splat(gv, zero16, h + H)
                    rows[_b][e0, pl.ds(h * HD, HD)] = (
                        rows[_b][e0, pl.ds(h * HD, HD)] * g0)
                    rows[_b][e0 + 1, pl.ds(h * HD, HD)] = (
                        rows[_b][e0 + 1, pl.ds(h * HD, HD)] * g1)
                return carry2

            lax.fori_loop(0, SUB // 2, pair_body, 0, unroll=2)
            sd[b] = pltpu.async_copy(rows[b], acc.at[tgt_v.at[g]],
                                     ssem[b], add=True)
        sd[0].wait()
        sd[1].wait()
        return carry

    lax.fori_loop(0, nchunk, chunk_body, 0)
    plsc.subcore_barrier()
    pltpu.sync_copy(acc.at[pl.ds(r0, NR)], out_hbm.at[c, pl.ds(r0, NR)])

    @pl.when(s == NS - 1)
    def _write_tail():
        pltpu.sync_copy(acc.at[pl.ds(NS * NR, 16)],
                        out_hbm.at[c, pl.ds(NS * NR, 16)])


def _make_sc_call():
    return functools.partial(
        pl.kernel,
        out_type=jax.ShapeDtypeStruct((NC, N_NODES, D), jnp.float32),
        mesh=plsc.VectorSubcoreMesh(core_axis_name="c", subcore_axis_name="s",
                                num_cores=NC, num_subcores=NS),
        scratch_types=[
        pltpu.VMEM_SHARED((N_NODES, D), jnp.float32),
        pltpu.VMEM((CH_ROWS, 128), jnp.int32),
        pltpu.VMEM((CH_ROWS, 128), jnp.int32),
        pltpu.VMEM((CH * H,), jnp.float32),
        pltpu.VMEM((SUB, D), jnp.float32),
        pltpu.VMEM((SUB, D), jnp.float32),
        pltpu.SemaphoreType.DMA,
        pltpu.SemaphoreType.DMA,
        pltpu.SemaphoreType.DMA,
        pltpu.SemaphoreType.DMA,
        pltpu.SemaphoreType.DMA,
        ],
    )(_sc_body)


_SC_CALL_CACHE = []


def _sc_call(*args):
    if not _SC_CALL_CACHE:
        _SC_CALL_CACHE.append(_make_sc_call())
    return _SC_CALL_CACHE[0](*args)


def kernel(x, edge_index, edge_attr, Wn, bn, We, be):
    x_proj, gates_p, src2, tgt2 = pl.pallas_call(
        _prep_body,
        grid=(40,),
        in_specs=[
            pl.BlockSpec((N_NODES, D), lambda i: (0, 0)),
            pl.BlockSpec((D, D), lambda i: (0, 0)),
            pl.BlockSpec((1, D), lambda i: (0, 0)),
            pl.BlockSpec((EB, 16), lambda i: (i, 0)),
            pl.BlockSpec((16, H), lambda i: (0, 0)),
            pl.BlockSpec((1, H), lambda i: (0, 0)),
            pl.BlockSpec((2, EB), lambda i: (0, i)),
        ],
        out_specs=[
            pl.BlockSpec((N_NODES + 8, D), lambda i: (0, 0)),
            pl.BlockSpec((EB, H), lambda i: (i, 0)),
            pl.BlockSpec((EB // 128, 128), lambda i: (i, 0)),
            pl.BlockSpec((EB // 128, 128), lambda i: (i, 0)),
        ],
        out_shape=[
            jax.ShapeDtypeStruct((N_NODES + 8, D), jnp.float32),
            jax.ShapeDtypeStruct((E_PAD, H), jnp.float32),
            jax.ShapeDtypeStruct((E_PAD // 128, 128), jnp.int32),
            jax.ShapeDtypeStruct((E_PAD // 128, 128), jnp.int32),
        ],
    )(x, Wn, bn.reshape(1, D), edge_attr, We, be.reshape(1, H),
      edge_index.astype(jnp.int32))

    parts = _sc_call(x_proj, src2, tgt2, gates_p.reshape(E_PAD * H))

    out = pl.pallas_call(
        _sum_body,
        out_shape=jax.ShapeDtypeStruct((N_NODES, D), jnp.float32),
    )(parts[0], parts[1])
    return out
